# Initial kernel scaffold; baseline (speedup 1.0000x reference)
import jax
import jax.numpy as jnp
from jax.experimental import pallas as pl


def kernel(emb_user, emb_item, edge_index, W_self, W_msg, b):
    raise NotImplementedError("write your pallas kernel here")


# speedup vs baseline: 3.6890x; 3.6890x over previous
import functools

import jax
import jax.numpy as jnp
from jax import lax
from jax.experimental import pallas as pl
from jax.experimental.pallas import tpu as pltpu
from jax.experimental.pallas import tpu_sc as plsc

N_NODES = 10000
N_EDGES = 160000
D_FEAT = 256
H = 128
NSUB = 16
CHUNK = 128
CHUNKS = 79
E_PAD = NSUB * CHUNKS * CHUNK
N_PAD = 10240
STRIPE = N_PAD // NSUB
STRIPE_CHUNK = 128


def _sc_agg_body(u0, u1, eidx, agg0, agg1, acc, idx_v, rows_v, sem):
    c = lax.axis_index("c")
    s = lax.axis_index("s")

    def zrow(i, carry):
        for j in range(H // 16):
            rows_v[i, pl.ds(j * 16, 16)] = jnp.zeros((16,), jnp.float32)
        return carry
    lax.fori_loop(0, STRIPE_CHUNK, zrow, 0)

    base_row = s * STRIPE
    for k in range(STRIPE // STRIPE_CHUNK):
        pltpu.sync_copy(rows_v, acc.at[pl.ds(base_row + k * STRIPE_CHUNK,
                                             STRIPE_CHUNK)])
    plsc.subcore_barrier()

    def edge_loop(u_ref):
        def body(j, carry):
            pltpu.sync_copy(eidx.at[s, j], idx_v)
            pltpu.async_copy(u_ref.at[idx_v.at[0]], rows_v, sem).wait()
            pltpu.sync_copy(rows_v, acc.at[idx_v.at[1]], add=True)
            return carry
        lax.fori_loop(0, CHUNKS, body, 0)

    pl.when(c == 0)(lambda: edge_loop(u0))
    pl.when(c == 1)(lambda: edge_loop(u1))

    plsc.subcore_barrier()

    def writeout(agg_ref):
        for k in range(STRIPE // STRIPE_CHUNK):
            rows = pl.ds(base_row + k * STRIPE_CHUNK, STRIPE_CHUNK)
            pltpu.sync_copy(acc.at[rows], rows_v)
            pltpu.sync_copy(rows_v, agg_ref.at[rows])

    pl.when(c == 0)(lambda: writeout(agg0))
    pl.when(c == 1)(lambda: writeout(agg1))


_sc_agg = functools.partial(
    pl.kernel,
    out_type=(jax.ShapeDtypeStruct((N_PAD, H), jnp.float32),
              jax.ShapeDtypeStruct((N_PAD, H), jnp.float32)),
    mesh=plsc.VectorSubcoreMesh(core_axis_name="c", subcore_axis_name="s"),
    scratch_types=[
        pltpu.VMEM_SHARED((N_PAD, H), jnp.float32),
        pltpu.VMEM((2, CHUNK), jnp.int32),
        pltpu.VMEM((CHUNK, H), jnp.float32),
        pltpu.SemaphoreType.DMA,
    ],
)(_sc_agg_body)


def _tc_body(xi_ref, a0_ref, a1_ref, ws_ref, wm_ref, b_ref, out_ref):
    f32 = jnp.float32
    hi = jax.lax.Precision.HIGHEST
    wm = wm_ref[...]
    acc = jnp.dot(xi_ref[...], ws_ref[...], preferred_element_type=f32,
                  precision=hi)
    acc += jnp.dot(a0_ref[...], wm[:H, :], preferred_element_type=f32,
                   precision=hi)
    acc += jnp.dot(a1_ref[...], wm[H:, :], preferred_element_type=f32,
                   precision=hi)
    out_ref[...] = acc + b_ref[...]


_TC_ROWS = 1000


def _tc_combine(x_item, agg0, agg1, W_self, W_msg, b2):
    return pl.pallas_call(
        _tc_body,
        grid=(N_NODES // _TC_ROWS,),
        in_specs=[
            pl.BlockSpec((_TC_ROWS, D_FEAT), lambda i: (i, 0)),
            pl.BlockSpec((_TC_ROWS, H), lambda i: (i, 0)),
            pl.BlockSpec((_TC_ROWS, H), lambda i: (i, 0)),
            pl.BlockSpec((D_FEAT, D_FEAT), lambda i: (0, 0)),
            pl.BlockSpec((D_FEAT, D_FEAT), lambda i: (0, 0)),
            pl.BlockSpec((1, D_FEAT), lambda i: (0, 0)),
        ],
        out_specs=pl.BlockSpec((_TC_ROWS, D_FEAT), lambda i: (i, 0)),
        out_shape=jax.ShapeDtypeStruct((N_NODES, D_FEAT), jnp.float32),
    )(x_item, agg0, agg1, W_self, W_msg, b2)


def kernel(emb_user, emb_item, edge_index, W_self, W_msg, b):
    src = edge_index[0]
    dst = edge_index[1]
    pad = E_PAD - N_EDGES
    src_p = jnp.concatenate([src, jnp.zeros((pad,), jnp.int32)])
    dst_p = jnp.concatenate([dst, jnp.full((pad,), N_NODES, jnp.int32)])
    eidx = jnp.stack([src_p.reshape(NSUB, CHUNKS, CHUNK),
                      dst_p.reshape(NSUB, CHUNKS, CHUNK)], axis=2)
    u0 = emb_user[:, :H]
    u1 = emb_user[:, H:]

    agg0, agg1 = _sc_agg(u0, u1, eidx)

    out_item = _tc_combine(emb_item, agg0, agg1, W_self, W_msg,
                           b.reshape(1, D_FEAT))
    return (emb_user, out_item)
